# baseline (speedup 1.0000x reference)
import jax
import jax.numpy as jnp
from jax.experimental import pallas as pl

_OUT = 7
_G = 2
_SCALE = 0.0625


def _make_kernel(B, C, H, W, R):
    S = _OUT * _G

    def _weights(c, size):
        cc = jnp.maximum(c, 0.0)
        lo = jnp.floor(cc)
        at_edge = lo >= (size - 1)
        lo = jnp.where(at_edge, float(size - 1), lo)
        hi = jnp.where(at_edge, lo, lo + 1.0)
        l = jnp.where(at_edge, 0.0, cc - lo)
        h = 1.0 - l
        valid = ((c >= -1.0) & (c <= float(size))).astype(jnp.float32)
        return lo, hi, l * valid, h * valid

    def _kern(feat_ref, rois_ref, out_ref):
        rois = rois_ref[:]
        b = rois[:, 0:1]
        x1 = rois[:, 1:2] * _SCALE
        y1 = rois[:, 2:3] * _SCALE
        x2 = rois[:, 3:4] * _SCALE
        y2 = rois[:, 4:5] * _SCALE

        bin_h = jnp.maximum(y2 - y1, 1.0) / _OUT
        bin_w = jnp.maximum(x2 - x1, 1.0) / _OUT

        jj = (jax.lax.broadcasted_iota(jnp.int32, (R, S), 1)
              .astype(jnp.float32) + 0.5) / _G
        ys = y1 + jj * bin_h
        xs = x1 + jj * bin_w

        ylo, yhi, ly, hy = _weights(ys, H)
        xlo, xhi, lx, hx = _weights(xs, W)

        yoff = b * H
        ylo_i = (ylo + yoff).astype(jnp.int32)[:, :, None]
        yhi_i = (yhi + yoff).astype(jnp.int32)[:, :, None]
        idy = jax.lax.broadcasted_iota(jnp.int32, (R, S, B * H), 2)
        Ay = (jnp.where(idy == ylo_i, hy[:, :, None], 0.0) +
              jnp.where(idy == yhi_i, ly[:, :, None], 0.0))
        Ay = Ay.reshape(R, _OUT, _G, B * H).sum(axis=2) * (1.0 / _G)

        xlo_i = xlo.astype(jnp.int32)[:, :, None]
        xhi_i = xhi.astype(jnp.int32)[:, :, None]
        idx = jax.lax.broadcasted_iota(jnp.int32, (R, S, W), 2)
        Ax = (jnp.where(idx == xlo_i, hx[:, :, None], 0.0) +
              jnp.where(idx == xhi_i, lx[:, :, None], 0.0))
        Ax = Ax.reshape(R, _OUT, _G, W).sum(axis=2) * (1.0 / _G)

        t1 = jnp.dot(Ay.reshape(R * _OUT, B * H), feat_ref[:],
                     preferred_element_type=jnp.float32)
        t1 = t1.reshape(R * _OUT, W, C)
        axb = jnp.broadcast_to(Ax[:, None], (R, _OUT, _OUT, W))
        axb = axb.reshape(R * _OUT, _OUT, W)
        o = jax.lax.dot_general(
            axb, t1, (((2,), (1,)), ((0,), (0,))),
            preferred_element_type=jnp.float32)
        o = o.reshape(R, _OUT * _OUT, C)
        out_ref[...] = jnp.transpose(o, (0, 2, 1))

    return _kern


def kernel(input, rois):
    B, C, H, W = input.shape
    N = rois.shape[0]
    R = 64
    feat_r = jnp.transpose(input, (0, 2, 3, 1)).reshape(B * H, W * C)
    out = pl.pallas_call(
        _make_kernel(B, C, H, W, R),
        grid=(N // R,),
        in_specs=[
            pl.BlockSpec((B * H, C * W), lambda i: (0, 0)),
            pl.BlockSpec((R, 5), lambda i: (i, 0)),
        ],
        out_specs=pl.BlockSpec((R, C, _OUT * _OUT), lambda i: (i, 0, 0)),
        out_shape=jax.ShapeDtypeStruct((N, C, _OUT * _OUT), jnp.float32),
    )(feat_r, rois)
    return out.reshape(N, C, _OUT, _OUT)


# speedup vs baseline: 1.0035x; 1.0035x over previous
import jax
import jax.numpy as jnp
from jax.experimental import pallas as pl

_OUT = 7
_G = 2
_SCALE = 0.0625


def _make_kernel(B, C, H, W, R):
    S = _OUT * _G

    def _weights(c, size):
        cc = jnp.maximum(c, 0.0)
        lo = jnp.floor(cc)
        at_edge = lo >= (size - 1)
        lo = jnp.where(at_edge, float(size - 1), lo)
        hi = jnp.where(at_edge, lo, lo + 1.0)
        l = jnp.where(at_edge, 0.0, cc - lo)
        h = 1.0 - l
        valid = ((c >= -1.0) & (c <= float(size))).astype(jnp.float32)
        return lo, hi, l * valid, h * valid

    def _kern(feat_ref, rois_ref, out_ref):
        rois = rois_ref[:]
        b = rois[:, 0:1]
        x1 = rois[:, 1:2] * _SCALE
        y1 = rois[:, 2:3] * _SCALE
        x2 = rois[:, 3:4] * _SCALE
        y2 = rois[:, 4:5] * _SCALE

        bin_h = jnp.maximum(y2 - y1, 1.0) / _OUT
        bin_w = jnp.maximum(x2 - x1, 1.0) / _OUT

        jj = (jax.lax.broadcasted_iota(jnp.int32, (R, S), 1)
              .astype(jnp.float32) + 0.5) / _G
        ys = y1 + jj * bin_h
        xs = x1 + jj * bin_w

        ylo, yhi, ly, hy = _weights(ys, H)
        xlo, xhi, lx, hx = _weights(xs, W)

        yoff = b * H
        ylo_i = (ylo + yoff).astype(jnp.int32)[:, :, None]
        yhi_i = (yhi + yoff).astype(jnp.int32)[:, :, None]
        idy = jax.lax.broadcasted_iota(jnp.int32, (R, S, B * H), 2)
        Ay = (jnp.where(idy == ylo_i, hy[:, :, None], 0.0) +
              jnp.where(idy == yhi_i, ly[:, :, None], 0.0))
        Ay = Ay.reshape(R, _OUT, _G, B * H).sum(axis=2) * (1.0 / _G)

        xlo_i = xlo.astype(jnp.int32)[:, :, None]
        xhi_i = xhi.astype(jnp.int32)[:, :, None]
        idx = jax.lax.broadcasted_iota(jnp.int32, (R, S, W), 2)
        Ax = (jnp.where(idx == xlo_i, hx[:, :, None], 0.0) +
              jnp.where(idx == xhi_i, lx[:, :, None], 0.0))
        Ax = Ax.reshape(R, _OUT, _G, W).sum(axis=2) * (1.0 / _G)

        t1 = jnp.dot(Ay.reshape(R * _OUT, B * H).astype(jnp.bfloat16),
                     feat_ref[:],
                     preferred_element_type=jnp.float32)
        t1 = t1.reshape(R * _OUT, W, C)
        axb = jnp.broadcast_to(Ax[:, None], (R, _OUT, _OUT, W))
        axb = axb.reshape(R * _OUT, _OUT, W)
        o = jax.lax.dot_general(
            axb, t1, (((2,), (1,)), ((0,), (0,))),
            preferred_element_type=jnp.float32)
        o = o.reshape(R, _OUT * _OUT, C)
        out_ref[...] = jnp.transpose(o, (0, 2, 1))

    return _kern


def kernel(input, rois):
    B, C, H, W = input.shape
    N = rois.shape[0]
    R = 64
    feat_r = jnp.transpose(input, (0, 2, 3, 1)).reshape(B * H, W * C)
    feat_r = feat_r.astype(jnp.bfloat16)
    out = pl.pallas_call(
        _make_kernel(B, C, H, W, R),
        grid=(N // R,),
        in_specs=[
            pl.BlockSpec((B * H, C * W), lambda i: (0, 0)),
            pl.BlockSpec((R, 5), lambda i: (i, 0)),
        ],
        out_specs=pl.BlockSpec((R, C, _OUT * _OUT), lambda i: (i, 0, 0)),
        out_shape=jax.ShapeDtypeStruct((N, C, _OUT * _OUT), jnp.float32),
    )(feat_r, rois)
    return out.reshape(N, C, _OUT, _OUT)


# speedup vs baseline: 1.0708x; 1.0671x over previous
import jax
import jax.numpy as jnp
from jax.experimental import pallas as pl
from jax.experimental.pallas import tpu as pltpu

_OUT = 7
_G = 2
_SCALE = 0.0625


def _make_kernel(B, C, H, W, R):
    S = _OUT * _G

    def _weights(c, size):
        cc = jnp.maximum(c, 0.0)
        lo = jnp.floor(cc)
        at_edge = lo >= (size - 1)
        lo = jnp.where(at_edge, float(size - 1), lo)
        hi = jnp.where(at_edge, lo, lo + 1.0)
        l = jnp.where(at_edge, 0.0, cc - lo)
        h = 1.0 - l
        valid = ((c >= -1.0) & (c <= float(size))).astype(jnp.float32)
        return lo, hi, l * valid, h * valid

    def _kern(feat_ref, rois_ref, out_ref, ft_ref):
        @pl.when(pl.program_id(0) == 0)
        def _init():
            for b in range(B):
                fb = feat_ref[b * C:(b + 1) * C, :]
                fb = jnp.transpose(fb, (1, 0))
                ft_ref[b * H:(b + 1) * H, :] = (
                    fb.astype(jnp.bfloat16).reshape(H, W * C))

        rois = rois_ref[:]
        b = rois[:, 0:1]
        x1 = rois[:, 1:2] * _SCALE
        y1 = rois[:, 2:3] * _SCALE
        x2 = rois[:, 3:4] * _SCALE
        y2 = rois[:, 4:5] * _SCALE

        bin_h = jnp.maximum(y2 - y1, 1.0) / _OUT
        bin_w = jnp.maximum(x2 - x1, 1.0) / _OUT

        jj = (jax.lax.broadcasted_iota(jnp.int32, (R, S), 1)
              .astype(jnp.float32) + 0.5) / _G
        ys = y1 + jj * bin_h
        xs = x1 + jj * bin_w

        ylo, yhi, ly, hy = _weights(ys, H)
        xlo, xhi, lx, hx = _weights(xs, W)

        yoff = b * H
        ylo_i = (ylo + yoff).astype(jnp.int32)[:, :, None]
        yhi_i = (yhi + yoff).astype(jnp.int32)[:, :, None]
        idy = jax.lax.broadcasted_iota(jnp.int32, (R, S, B * H), 2)
        Ay = (jnp.where(idy == ylo_i, hy[:, :, None], 0.0) +
              jnp.where(idy == yhi_i, ly[:, :, None], 0.0))
        Ay = Ay.reshape(R, _OUT, _G, B * H).sum(axis=2) * (1.0 / _G)

        xlo_i = xlo.astype(jnp.int32)[:, :, None]
        xhi_i = xhi.astype(jnp.int32)[:, :, None]
        idx = jax.lax.broadcasted_iota(jnp.int32, (R, S, W), 2)
        Ax = (jnp.where(idx == xlo_i, hx[:, :, None], 0.0) +
              jnp.where(idx == xhi_i, lx[:, :, None], 0.0))
        Ax = Ax.reshape(R, _OUT, _G, W).sum(axis=2) * (1.0 / _G)

        t1 = jnp.dot(Ay.reshape(R * _OUT, B * H).astype(jnp.bfloat16),
                     ft_ref[:],
                     preferred_element_type=jnp.float32)
        t1 = t1.astype(jnp.bfloat16).reshape(R * _OUT, W, C)
        axb = jnp.broadcast_to(Ax[:, None], (R, _OUT, _OUT, W))
        axb = axb.reshape(R * _OUT, _OUT, W).astype(jnp.bfloat16)
        o = jax.lax.dot_general(
            axb, t1, (((2,), (1,)), ((0,), (0,))),
            preferred_element_type=jnp.float32)
        o = o.reshape(R, _OUT * _OUT, C)
        out_ref[...] = jnp.transpose(o, (0, 2, 1))

    return _kern


def kernel(input, rois):
    B, C, H, W = input.shape
    N = rois.shape[0]
    R = 64
    feat2 = input.reshape(B * C, H * W)
    out = pl.pallas_call(
        _make_kernel(B, C, H, W, R),
        grid=(N // R,),
        in_specs=[
            pl.BlockSpec((B * C, H * W), lambda i: (0, 0)),
            pl.BlockSpec((R, 5), lambda i: (i, 0)),
        ],
        out_specs=pl.BlockSpec((R, C, _OUT * _OUT), lambda i: (i, 0, 0)),
        out_shape=jax.ShapeDtypeStruct((N, C, _OUT * _OUT), jnp.float32),
        scratch_shapes=[pltpu.VMEM((B * H, W * C), jnp.bfloat16)],
    )(feat2, rois)
    return out.reshape(N, C, _OUT, _OUT)


# speedup vs baseline: 1.8045x; 1.6851x over previous
import jax
import jax.numpy as jnp
from jax.experimental import pallas as pl
from jax.experimental.pallas import tpu as pltpu

_OUT = 7
_G = 2
_SCALE = 0.0625


def _make_kernel(B, C, H, W, R):
    S = _OUT * _G

    def _weights(c, size):
        cc = jnp.maximum(c, 0.0)
        lo = jnp.floor(cc)
        at_edge = lo >= (size - 1)
        lo = jnp.where(at_edge, float(size - 1), lo)
        hi = jnp.where(at_edge, lo, lo + 1.0)
        l = jnp.where(at_edge, 0.0, cc - lo)
        h = 1.0 - l
        valid = ((c >= -1.0) & (c <= float(size))).astype(jnp.float32)
        return lo, hi, l * valid, h * valid

    def _kern(feat_ref, rois_ref, out_ref, ft_ref):
        @pl.when(pl.program_id(0) == 0)
        def _init():
            for b in range(B):
                fb = feat_ref[:, b * C:(b + 1) * C]
                ft_ref[b * H:(b + 1) * H, :] = (
                    fb.astype(jnp.bfloat16).reshape(H, W * C))

        rois = rois_ref[:]
        b = rois[:, 0:1]
        x1 = rois[:, 1:2] * _SCALE
        y1 = rois[:, 2:3] * _SCALE
        x2 = rois[:, 3:4] * _SCALE
        y2 = rois[:, 4:5] * _SCALE

        bin_h = jnp.maximum(y2 - y1, 1.0) / _OUT
        bin_w = jnp.maximum(x2 - x1, 1.0) / _OUT

        jj = (jax.lax.broadcasted_iota(jnp.int32, (R, S), 1)
              .astype(jnp.float32) + 0.5) / _G
        ys = y1 + jj * bin_h
        xs = x1 + jj * bin_w

        ylo, yhi, ly, hy = _weights(ys, H)
        xlo, xhi, lx, hx = _weights(xs, W)

        yoff = b * H
        ylo_i = (ylo + yoff).astype(jnp.int32)[:, :, None]
        yhi_i = (yhi + yoff).astype(jnp.int32)[:, :, None]
        idy = jax.lax.broadcasted_iota(jnp.int32, (R, S, B * H), 2)
        Ay = (jnp.where(idy == ylo_i, hy[:, :, None], 0.0) +
              jnp.where(idy == yhi_i, ly[:, :, None], 0.0))
        Ay = Ay.reshape(R, _OUT, _G, B * H).sum(axis=2) * (1.0 / _G)

        xlo_i = xlo.astype(jnp.int32)[:, :, None]
        xhi_i = xhi.astype(jnp.int32)[:, :, None]
        idx = jax.lax.broadcasted_iota(jnp.int32, (R, S, W), 2)
        Ax = (jnp.where(idx == xlo_i, hx[:, :, None], 0.0) +
              jnp.where(idx == xhi_i, lx[:, :, None], 0.0))
        Ax = Ax.reshape(R, _OUT, _G, W).sum(axis=2) * (1.0 / _G)

        t1 = jnp.dot(Ay.reshape(R * _OUT, B * H).astype(jnp.bfloat16),
                     ft_ref[:],
                     preferred_element_type=jnp.float32)
        t1 = t1.astype(jnp.bfloat16).reshape(R * _OUT, W, C)
        axb = jnp.broadcast_to(Ax[:, None], (R, _OUT, _OUT, W))
        axb = axb.reshape(R * _OUT, _OUT, W).astype(jnp.bfloat16)
        o = jax.lax.dot_general(
            axb, t1, (((2,), (1,)), ((0,), (0,))),
            preferred_element_type=jnp.float32)
        o = jnp.transpose(o.reshape(R, _OUT, _OUT, C), (1, 2, 0, 3))
        out_ref[...] = o.reshape(_OUT * _OUT, R, C)

    return _kern


def kernel(input, rois):
    B, C, H, W = input.shape
    N = rois.shape[0]
    R = 64
    feat2 = jnp.transpose(input, (2, 3, 0, 1)).reshape(H * W, B * C)
    out = pl.pallas_call(
        _make_kernel(B, C, H, W, R),
        grid=(N // R,),
        in_specs=[
            pl.BlockSpec((H * W, B * C), lambda i: (0, 0)),
            pl.BlockSpec((R, 5), lambda i: (i, 0)),
        ],
        out_specs=pl.BlockSpec((_OUT * _OUT, R, C), lambda i: (0, i, 0)),
        out_shape=jax.ShapeDtypeStruct((_OUT * _OUT, N, C), jnp.float32),
        scratch_shapes=[pltpu.VMEM((B * H, W * C), jnp.bfloat16)],
    )(feat2, rois)
    return jnp.transpose(out.reshape(_OUT, _OUT, N, C), (2, 3, 0, 1))
